# Initial kernel scaffold; baseline (speedup 1.0000x reference)
import jax
import jax.numpy as jnp
from jax.experimental import pallas as pl


def kernel(tokens, table):
    raise NotImplementedError("write your pallas kernel here")


# speedup vs baseline: 1.4166x; 1.4166x over previous
import functools
import math

import jax
import jax.numpy as jnp
from jax import lax
from jax.experimental import pallas as pl
from jax.experimental.pallas import tpu as pltpu
from jax.experimental.pallas import tpu_sc as plsc

EMB = 32
SCALE = math.sqrt(float(EMB))
NUM_CORES = 2
NUM_SUBCORES = 16
NW = NUM_CORES * NUM_SUBCORES


def _pick_chunk(rows_per_w: int, cap: int = 1600) -> int:
    for c in range(min(cap, rows_per_w), 0, -1):
        if rows_per_w % c == 0:
            return c
    return rows_per_w


@functools.lru_cache(maxsize=None)
def _build(B: int, D: int):
    rows_per_w = B // NW
    chunk = _pick_chunk(rows_per_w)
    nchunk = rows_per_w // chunk
    mesh = plsc.VectorSubcoreMesh(core_axis_name="c", subcore_axis_name="s")

    @functools.partial(
        pl.kernel,
        mesh=mesh,
        out_type=jax.ShapeDtypeStruct((B, D), jnp.float32),
        scratch_types=[
            pltpu.VMEM((chunk,), jnp.int32),
            pltpu.VMEM((chunk, D), jnp.float32),
            pltpu.SemaphoreType.DMA,
        ],
        compiler_params=pltpu.CompilerParams(use_tc_tiling_on_sc=False),
    )
    def emb_kernel(tokens_hbm, table_hbm, out_hbm, idx_v, rows_v, sem):
        wid = lax.axis_index("s") * NUM_CORES + lax.axis_index("c")
        base0 = wid * rows_per_w

        def chunk_body(ci, carry):
            base = base0 + ci * chunk
            pltpu.sync_copy(tokens_hbm.at[pl.ds(base, chunk)], idx_v)
            pltpu.async_copy(table_hbm.at[idx_v], rows_v, sem).wait()

            def scale_body(i, c):
                for j in range(D // 16):
                    sl = pl.ds(j * 16, 16)
                    rows_v[i, sl] = rows_v[i, sl] * SCALE
                return c

            lax.fori_loop(0, chunk, scale_body, 0, unroll=4)
            pltpu.sync_copy(rows_v, out_hbm.at[pl.ds(base, chunk)])
            return carry

        lax.fori_loop(0, nchunk, chunk_body, 0)

    return emb_kernel


def kernel(tokens, table):
    B = int(tokens.size)
    D = int(table.shape[1])
    flat = tokens.reshape((B,)).astype(jnp.int32)
    out = _build(B, D)(flat, table)
    return out.reshape(tuple(tokens.shape) + (D,))


# speedup vs baseline: 1.4786x; 1.0438x over previous
import functools
import math

import jax
import jax.numpy as jnp
from jax import lax
from jax.experimental import pallas as pl
from jax.experimental.pallas import tpu as pltpu
from jax.experimental.pallas import tpu_sc as plsc

EMB = 32
SCALE = math.sqrt(float(EMB))
NUM_CORES = 2
NUM_SUBCORES = 16
NW = NUM_CORES * NUM_SUBCORES


def _pick_chunk(rows_per_w: int, cap: int = 1600) -> int:
    for c in range(min(cap, rows_per_w), 0, -1):
        if rows_per_w % c == 0:
            return c
    return rows_per_w


@functools.lru_cache(maxsize=None)
def _build(B: int, D: int):
    rows_per_w = B // NW
    chunk = _pick_chunk(rows_per_w)
    nchunk = rows_per_w // chunk
    mesh = plsc.VectorSubcoreMesh(core_axis_name="c", subcore_axis_name="s")

    @functools.partial(
        pl.kernel,
        mesh=mesh,
        out_type=jax.ShapeDtypeStruct((B, D), jnp.float32),
        scratch_types=[
            pltpu.VMEM((chunk,), jnp.int32),
            pltpu.VMEM((chunk,), jnp.int32),
            pltpu.VMEM((chunk, D), jnp.float32),
            pltpu.VMEM((chunk, D), jnp.float32),
            pltpu.SemaphoreType.DMA,
            pltpu.SemaphoreType.DMA,
            pltpu.SemaphoreType.DMA,
            pltpu.SemaphoreType.DMA,
            pltpu.SemaphoreType.DMA,
            pltpu.SemaphoreType.DMA,
        ],
        compiler_params=pltpu.CompilerParams(use_tc_tiling_on_sc=False),
    )
    def emb_kernel(tokens_hbm, table_hbm, out_hbm,
                   idx0, idx1, rows0, rows1,
                   isem0, isem1, gsem0, gsem1, ssem0, ssem1):
        idx = (idx0, idx1)
        rows = (rows0, rows1)
        isem = (isem0, isem1)
        gsem = (gsem0, gsem1)
        ssem = (ssem0, ssem1)
        wid = lax.axis_index("s") * NUM_CORES + lax.axis_index("c")
        base0 = wid * rows_per_w

        def tok_slice(ci):
            return tokens_hbm.at[pl.ds(base0 + ci * chunk, chunk)]

        def out_slice(ci):
            return out_hbm.at[pl.ds(base0 + ci * chunk, chunk)]

        def scale_chunk(b):
            def scale_body(i, c):
                for j in range(D // 16):
                    sl = pl.ds(j * 16, 16)
                    rows[b][i, sl] = rows[b][i, sl] * SCALE
                return c

            lax.fori_loop(0, chunk, scale_body, 0, unroll=8)

        pltpu.async_copy(tok_slice(0), idx0, isem0)
        if nchunk > 1:
            pltpu.async_copy(tok_slice(1), idx1, isem1)
        pltpu.make_async_copy(tok_slice(0), idx0, isem0).wait()
        pltpu.async_copy(table_hbm.at[idx0], rows0, gsem0)

        for ci in range(nchunk):
            b = ci & 1
            b1 = b ^ 1
            pltpu.make_async_copy(table_hbm.at[idx[b]], rows[b], gsem[b]).wait()
            if ci + 2 < nchunk:
                pltpu.async_copy(tok_slice(ci + 2), idx[b], isem[b])
            if ci + 1 < nchunk:
                pltpu.make_async_copy(tok_slice(ci + 1), idx[b1], isem[b1]).wait()
                if ci >= 1:
                    pltpu.make_async_copy(rows[b1], out_slice(ci - 1), ssem[b1]).wait()
                pltpu.async_copy(table_hbm.at[idx[b1]], rows[b1], gsem[b1])
            scale_chunk(b)
            pltpu.async_copy(rows[b], out_slice(ci), ssem[b])

        lb = (nchunk - 1) & 1
        pltpu.make_async_copy(rows[lb], out_slice(nchunk - 1), ssem[lb]).wait()
        if nchunk > 1:
            pltpu.make_async_copy(rows[lb ^ 1], out_slice(nchunk - 2), ssem[lb ^ 1]).wait()

    return emb_kernel


def kernel(tokens, table):
    B = int(tokens.size)
    D = int(table.shape[1])
    flat = tokens.reshape((B,)).astype(jnp.int32)
    out = _build(B, D)(flat, table)
    return out.reshape(tuple(tokens.shape) + (D,))
